# baseline (speedup 1.0000x reference)
import functools

import jax
import jax.numpy as jnp
from jax import lax
from jax.experimental import pallas as pl
from jax.experimental.pallas import tpu as pltpu
from jax.experimental.pallas import tpu_sc as plsc

_K = 128
_NT = 16
_NC = 2
_NW = _NC * _NT


def _pre_body(x_ref, o_ref):
    x = x_ref[...]
    n, d = x.shape
    spars = 1.0 - jnp.sum((x != 0).astype(jnp.float32), axis=1, keepdims=True) / d
    ent = -jnp.sum(x * jnp.log(x + 1e-15), axis=1, keepdims=True)
    spars = (spars - jnp.min(spars)) / (jnp.max(spars) - jnp.min(spars))
    ent = (ent - jnp.min(ent)) / (jnp.max(ent) - jnp.min(ent))
    pad = jnp.zeros((n, o_ref.shape[1] - 2), jnp.float32)
    o_ref[...] = jnp.concatenate([spars, ent, pad], axis=1)


def _mid_body(n, px_ref, ps_ref, w1_ref, w2_ref, o_ref):
    ax = px_ref[0, :n, :] + px_ref[1, :n, :]
    as_ = ps_ref[0, :n, :] + ps_ref[1, :n, :]
    a = jnp.concatenate([ax, as_[:, :2]], axis=1)
    h = jnp.dot(a, w1_ref[...], preferred_element_type=jnp.float32)
    h = jnp.maximum(h, 0.0)
    d = h.shape[1]
    spars = 1.0 - jnp.sum((h != 0).astype(jnp.float32), axis=1, keepdims=True) / d
    ent = -jnp.sum(h * jnp.log(h + 1e-15), axis=1, keepdims=True)
    spars = (spars - jnp.min(spars)) / (jnp.max(spars) - jnp.min(spars))
    ent = (ent - jnp.min(ent)) / (jnp.max(ent) - jnp.min(ent))
    y = jnp.dot(h, w2_ref[:d, :], preferred_element_type=jnp.float32)
    y = y + spars * w2_ref[d:d + 1, :] + ent * w2_ref[d + 1:d + 2, :]
    o_ref[...] = y


def _post_body(n, p_ref, o_ref):
    a = p_ref[0, :n, :] + p_ref[1, :n, :]
    s = a - jnp.max(a, axis=1, keepdims=True)
    o_ref[...] = s - jnp.log(jnp.sum(jnp.exp(s), axis=1, keepdims=True))


def _make_agg(np_, ds, chb, ch_slow, k, nbuf):
    rows_per_tile = np_ // _NT
    nt = len(ds)
    ch_fast = 2 * chb - ch_slow
    ch = ch_fast
    mesh = plsc.VectorSubcoreMesh(core_axis_name="c", subcore_axis_name="s")

    @functools.partial(
        pl.kernel,
        out_type=tuple(
            jax.ShapeDtypeStruct((_NC, np_, d), jnp.float32) for d in ds),
        mesh=mesh,
        scratch_types=[
            pltpu.VMEM((ch, k), jnp.int32),
            pltpu.VMEM((ch, k), jnp.int32),
            [[pltpu.VMEM((k, d), jnp.float32) for _ in range(nbuf)]
             for d in ds],
            [pltpu.VMEM_SHARED((np_, d), jnp.float32) for d in ds],
            [[pltpu.SemaphoreType.DMA for _ in range(nbuf)] for _ in ds],
        ],
        compiler_params=pltpu.CompilerParams(use_tc_tiling_on_sc=False),
    )
    def agg(*refs):
        tables = refs[0:nt]
        rowc, colc = refs[nt], refs[nt + 1]
        zeros = refs[nt + 2:2 * nt + 2]
        outs = refs[2 * nt + 2:3 * nt + 2]
        rowv, colv = refs[3 * nt + 2], refs[3 * nt + 3]
        bufs, shareds, gsems = refs[3 * nt + 4], refs[3 * nt + 5], refs[3 * nt + 6]
        cid = lax.axis_index("c")
        sid = lax.axis_index("s")
        ch_eff = jnp.where(cid == 0, ch_fast, ch_slow)
        pltpu.sync_copy(rowc.at[cid, sid], rowv.at[pl.ds(0, chb)])
        pltpu.sync_copy(colc.at[cid, sid], colv.at[pl.ds(0, chb)])

        @pl.when(cid == 0)
        def _():
            pltpu.sync_copy(rowc.at[1, sid, pl.ds(ch_slow, chb - ch_slow)],
                            rowv.at[pl.ds(chb, chb - ch_slow)])
            pltpu.sync_copy(colc.at[1, sid, pl.ds(ch_slow, chb - ch_slow)],
                            colv.at[pl.ds(chb, chb - ch_slow)])
        r0 = sid * rows_per_tile
        for t in range(nt):
            pltpu.sync_copy(zeros[t].at[pl.ds(r0, rows_per_tile)],
                            shareds[t].at[pl.ds(r0, rows_per_tile)])
        plsc.subcore_barrier()

        for b in range(nbuf):
            for t in range(nt):
                pltpu.async_copy(tables[t].at[rowv.at[b]], bufs[t][b],
                                 gsems[t][b])

        def _ring(step, carry):
            i = step * nbuf
            for b in range(nbuf):
                j = i + b
                for t in range(nt):
                    pltpu.make_async_copy(tables[t].at[rowv.at[j]], bufs[t][b],
                                          gsems[t][b]).wait()
                    pltpu.sync_copy(bufs[t][b], shareds[t].at[colv.at[j]],
                                    add=True)

                    @pl.when(j + nbuf < ch_eff)
                    def _():
                        pltpu.async_copy(tables[t].at[rowv.at[j + nbuf]],
                                         bufs[t][b], gsems[t][b])
            return carry

        lax.fori_loop(0, ch_eff // nbuf, _ring, 0)
        plsc.subcore_barrier()
        for t in range(nt):
            pltpu.sync_copy(shareds[t].at[pl.ds(r0, rows_per_tile)],
                            outs[t].at[cid, pl.ds(r0, rows_per_tile)])

    return agg


def _edge_layout(row, col, n, e, chb, k):
    e_pad = _NW * k * chb
    pad = e_pad - e
    rowp = jnp.concatenate([row, jnp.zeros((pad,), jnp.int32)])
    colp = jnp.concatenate([col, jnp.full((pad,), n, jnp.int32)])
    return (rowp.reshape(_NC, _NT, chb, k), colp.reshape(_NC, _NT, chb, k))


def kernel(x, edge_index, W1, W2):
    n, f_in = x.shape
    e = edge_index.shape[1]
    hid = W1.shape[1]
    cls = W2.shape[1]
    np_ = ((n + _NT * 8 - 1) // (_NT * 8)) * _NT * 8
    k, nbuf1, nbuf2 = 64, 2, 4
    cht = (e + _NT * k - 1) // (_NT * k)
    chb = ((cht + 7) // 8) * 4
    ch_slow = max(((2 * chb * 32 // 100) // 4) * 4, 2 * chb - 216)

    row = edge_index[0]
    col = edge_index[1]
    rowc1, colc1 = _edge_layout(row, col, n, e, chb, k)
    rowc2, colc2 = rowc1, colc1
    zeros_x = jnp.zeros((np_, f_in), jnp.float32)
    zeros_s = jnp.zeros((np_, 8), jnp.float32)
    zeros2 = jnp.zeros((np_, cls), jnp.float32)

    stats = pl.pallas_call(
        _pre_body,
        out_shape=jax.ShapeDtypeStruct((n, 8), jnp.float32),
    )(x)
    part1x, part1s = _make_agg(np_, (f_in, 8), chb, ch_slow, k, nbuf1)(
        x, stats, rowc1, colc1, zeros_x, zeros_s)
    y2 = pl.pallas_call(
        functools.partial(_mid_body, n),
        out_shape=jax.ShapeDtypeStruct((n, cls), jnp.float32),
        compiler_params=pltpu.CompilerParams(vmem_limit_bytes=100 * 1024 * 1024),
    )(part1x, part1s, W1, W2)

    (part2,) = _make_agg(np_, (cls,), chb, ch_slow, k, nbuf2)(
        y2, rowc2, colc2, zeros2)
    out = pl.pallas_call(
        functools.partial(_post_body, n),
        out_shape=jax.ShapeDtypeStruct((n, cls), jnp.float32),
    )(part2)
    return out


# speedup vs baseline: 1.4532x; 1.4532x over previous
import functools

import jax
import jax.numpy as jnp
from jax import lax
from jax.experimental import pallas as pl
from jax.experimental.pallas import tpu as pltpu
from jax.experimental.pallas import tpu_sc as plsc

_K = 128
_NT = 16
_NC = 2
_NW = _NC * _NT


def _pre_body(x_ref, o_ref):
    x = x_ref[...]
    n, d = x.shape
    spars = 1.0 - jnp.sum((x != 0).astype(jnp.float32), axis=1, keepdims=True) / d
    ent = -jnp.sum(x * jnp.log(x + 1e-15), axis=1, keepdims=True)
    spars = (spars - jnp.min(spars)) / (jnp.max(spars) - jnp.min(spars))
    ent = (ent - jnp.min(ent)) / (jnp.max(ent) - jnp.min(ent))
    pad = jnp.zeros((n, o_ref.shape[1] - 2), jnp.float32)
    o_ref[...] = jnp.concatenate([spars, ent, pad], axis=1)


def _mid_body(n, px_ref, ps_ref, w1_ref, w2_ref, o_ref):
    ax = px_ref[0, :n, :] + px_ref[1, :n, :]
    as_ = ps_ref[0, :n, :] + ps_ref[1, :n, :]
    a = jnp.concatenate([ax, as_[:, :2]], axis=1)
    h = jnp.dot(a, w1_ref[...], preferred_element_type=jnp.float32)
    h = jnp.maximum(h, 0.0)
    d = h.shape[1]
    spars = 1.0 - jnp.sum((h != 0).astype(jnp.float32), axis=1, keepdims=True) / d
    ent = -jnp.sum(h * jnp.log(h + 1e-15), axis=1, keepdims=True)
    spars = (spars - jnp.min(spars)) / (jnp.max(spars) - jnp.min(spars))
    ent = (ent - jnp.min(ent)) / (jnp.max(ent) - jnp.min(ent))
    y = jnp.dot(h, w2_ref[:d, :], preferred_element_type=jnp.float32)
    y = y + spars * w2_ref[d:d + 1, :] + ent * w2_ref[d + 1:d + 2, :]
    o_ref[...] = y


def _post_body(n, p_ref, o_ref):
    a = p_ref[0, :n, :] + p_ref[1, :n, :]
    s = a - jnp.max(a, axis=1, keepdims=True)
    o_ref[...] = s - jnp.log(jnp.sum(jnp.exp(s), axis=1, keepdims=True))


def _make_agg(np_, ds, ch_by_core, k, nbuf):
    rows_per_tile = np_ // _NT
    nt = len(ds)
    ch = max(ch_by_core)
    mesh = plsc.VectorSubcoreMesh(core_axis_name="c", subcore_axis_name="s")

    @functools.partial(
        pl.kernel,
        out_type=tuple(
            jax.ShapeDtypeStruct((_NC, np_, d), jnp.float32) for d in ds),
        mesh=mesh,
        scratch_types=[
            pltpu.VMEM((ch, k), jnp.int32),
            pltpu.VMEM((ch, k), jnp.int32),
            [[pltpu.VMEM((k, d), jnp.float32) for _ in range(nbuf)]
             for d in ds],
            [pltpu.VMEM_SHARED((np_, d), jnp.float32) for d in ds],
            [[pltpu.SemaphoreType.DMA for _ in range(nbuf)] for _ in ds],
        ],
        compiler_params=pltpu.CompilerParams(use_tc_tiling_on_sc=False),
    )
    def agg(*refs):
        tables = refs[0:nt]
        rowc, colc = refs[nt], refs[nt + 1]
        zeros = refs[nt + 2:2 * nt + 2]
        outs = refs[2 * nt + 2:3 * nt + 2]
        rowv, colv = refs[3 * nt + 2], refs[3 * nt + 3]
        bufs, shareds, gsems = refs[3 * nt + 4], refs[3 * nt + 5], refs[3 * nt + 6]
        cid = lax.axis_index("c")
        sid = lax.axis_index("s")
        ch_eff = jnp.where(cid == 0, ch_by_core[0], ch_by_core[1])
        pltpu.sync_copy(rowc.at[cid, sid], rowv)
        pltpu.sync_copy(colc.at[cid, sid], colv)
        r0 = sid * rows_per_tile
        for t in range(nt):
            pltpu.sync_copy(zeros[t].at[pl.ds(r0, rows_per_tile)],
                            shareds[t].at[pl.ds(r0, rows_per_tile)])
        plsc.subcore_barrier()

        for b in range(nbuf):
            for t in range(nt):
                pltpu.async_copy(tables[t].at[rowv.at[b]], bufs[t][b],
                                 gsems[t][b])

        def _ring(step, carry):
            i = step * nbuf
            for b in range(nbuf):
                j = i + b
                for t in range(nt):
                    pltpu.make_async_copy(tables[t].at[rowv.at[j]], bufs[t][b],
                                          gsems[t][b]).wait()
                    pltpu.sync_copy(bufs[t][b], shareds[t].at[colv.at[j]],
                                    add=True)

                    @pl.when(j + nbuf < ch_eff)
                    def _():
                        pltpu.async_copy(tables[t].at[rowv.at[j + nbuf]],
                                         bufs[t][b], gsems[t][b])
            return carry

        lax.fori_loop(0, ch_eff // nbuf, _ring, 0)
        plsc.subcore_barrier()
        for t in range(nt):
            pltpu.sync_copy(shareds[t].at[pl.ds(r0, rows_per_tile)],
                            outs[t].at[cid, pl.ds(r0, rows_per_tile)])

    return agg


def _edge_layout(row, col, n, e, ch_by_core, k):
    ch0, ch1 = ch_by_core
    ch = max(ch0, ch1)
    e_pad = _NT * k * (ch0 + ch1)
    pad = e_pad - e
    rowp = jnp.concatenate([row, jnp.zeros((pad,), jnp.int32)])
    colp = jnp.concatenate([col, jnp.full((pad,), n, jnp.int32)])

    def _part(v, start, c, fill):
        blk = v[start:start + _NT * c * k].reshape(_NT, c, k)
        return jnp.pad(blk, ((0, 0), (0, ch - c), (0, 0)),
                       constant_values=fill)

    rowc = jnp.stack([_part(rowp, 0, ch0, 0),
                      _part(rowp, _NT * ch0 * k, ch1, 0)])
    colc = jnp.stack([_part(colp, 0, ch0, n),
                      _part(colp, _NT * ch0 * k, ch1, n)])
    return rowc, colc


def kernel(x, edge_index, W1, W2):
    n, f_in = x.shape
    e = edge_index.shape[1]
    hid = W1.shape[1]
    cls = W2.shape[1]
    np_ = ((n + _NT * 8 - 1) // (_NT * 8)) * _NT * 8
    k, nbuf1, nbuf2 = 64, 2, 4
    cht = (e + _NT * k - 1) // (_NT * k)
    ch_slow = ((cht * 32 // 100) // 4) * 4
    ch_fast = ((cht - ch_slow + 3) // 4) * 4
    ch_by_core = (ch_fast, ch_slow)

    row = edge_index[0]
    col = edge_index[1]
    rowc1, colc1 = _edge_layout(row, col, n, e, ch_by_core, k)
    rowc2, colc2 = rowc1, colc1
    zeros_x = jnp.zeros((np_, f_in), jnp.float32)
    zeros_s = jnp.zeros((np_, 8), jnp.float32)
    zeros2 = jnp.zeros((np_, cls), jnp.float32)

    stats = pl.pallas_call(
        _pre_body,
        out_shape=jax.ShapeDtypeStruct((n, 8), jnp.float32),
    )(x)
    part1x, part1s = _make_agg(np_, (f_in, 8), ch_by_core, k, nbuf1)(
        x, stats, rowc1, colc1, zeros_x, zeros_s)
    y2 = pl.pallas_call(
        functools.partial(_mid_body, n),
        out_shape=jax.ShapeDtypeStruct((n, cls), jnp.float32),
        compiler_params=pltpu.CompilerParams(vmem_limit_bytes=100 * 1024 * 1024),
    )(part1x, part1s, W1, W2)

    (part2,) = _make_agg(np_, (cls,), ch_by_core, k, nbuf2)(
        y2, rowc2, colc2, zeros2)
    out = pl.pallas_call(
        functools.partial(_post_body, n),
        out_shape=jax.ShapeDtypeStruct((n, cls), jnp.float32),
    )(part2)
    return out
